# Initial kernel scaffold; baseline (speedup 1.0000x reference)
import jax
import jax.numpy as jnp
from jax.experimental import pallas as pl


def kernel(x, table):
    raise NotImplementedError("write your pallas kernel here")


# speedup vs baseline: 4.0857x; 4.0857x over previous
import functools

import jax
import jax.numpy as jnp
from jax import lax
from jax.experimental import pallas as pl
from jax.experimental.pallas import tpu as pltpu
from jax.experimental.pallas import tpu_sc as plsc

_INFO = plsc.get_sparse_core_info()
_NC = _INFO.num_cores
_NS = _INFO.num_subcores
_NW = _NC * _NS

_VOCAB = 100000
_D = 64
_B = 4096 * 50
_CHUNK = 128
_ROWS_PER_W = _B // _NW
_CHUNKS_PER_W = _ROWS_PER_W // _CHUNK


def _make_gather():
    mesh = plsc.VectorSubcoreMesh(core_axis_name="c", subcore_axis_name="s")

    @functools.partial(
        pl.kernel,
        mesh=mesh,
        out_type=jax.ShapeDtypeStruct((_B, _D), jnp.float32),
        scratch_types=[
            pltpu.VMEM((_CHUNKS_PER_W, _CHUNK), jnp.int32),
            pltpu.VMEM((_CHUNK, _D), jnp.float32),
            pltpu.SemaphoreType.DMA,
        ],
        compiler_params=pltpu.CompilerParams(use_tc_tiling_on_sc=False),
    )
    def gather_kernel(idx_hbm, table_hbm, out_hbm, idx_v, rows_v, sem):
        wid = lax.axis_index("s") * _NC + lax.axis_index("c")
        base = wid * _ROWS_PER_W
        pltpu.sync_copy(idx_hbm.at[wid], idx_v)

        def step(j, carry):
            pltpu.async_copy(table_hbm.at[idx_v.at[j]], rows_v, sem).wait()
            pltpu.sync_copy(rows_v, out_hbm.at[pl.ds(base + j * _CHUNK, _CHUNK)])
            return carry

        lax.fori_loop(0, _CHUNKS_PER_W, step, 0)

    return gather_kernel


_gather = _make_gather()


@jax.jit
def kernel(x, table):
    batch, hist = x.shape
    idx = x.reshape(_NW, _CHUNKS_PER_W, _CHUNK)
    out = _gather(idx, table)
    return out.reshape(batch, hist, _D)


# speedup vs baseline: 4.6810x; 1.1457x over previous
import functools

import jax
import jax.numpy as jnp
from jax import lax
from jax.experimental import pallas as pl
from jax.experimental.pallas import tpu as pltpu
from jax.experimental.pallas import tpu_sc as plsc

_INFO = plsc.get_sparse_core_info()
_NC = _INFO.num_cores
_NS = _INFO.num_subcores
_NW = _NC * _NS

_D = 64
_B = 4096 * 50
_CHUNK = 128
_ROWS_PER_W = _B // _NW
_NCHUNK = _ROWS_PER_W // _CHUNK
_NBUF = 5
_NROUND = _NCHUNK // _NBUF


def _make_gather():
    mesh = plsc.VectorSubcoreMesh(core_axis_name="c", subcore_axis_name="s")

    @functools.partial(
        pl.kernel,
        mesh=mesh,
        out_type=jax.ShapeDtypeStruct((_B, _D), jnp.float32),
        scratch_types=(
            [pltpu.VMEM((_NCHUNK, _CHUNK), jnp.int32)]
            + [pltpu.VMEM((_CHUNK, _D), jnp.float32)] * _NBUF
            + [pltpu.SemaphoreType.DMA] * (2 * _NBUF)
        ),
        compiler_params=pltpu.CompilerParams(use_tc_tiling_on_sc=False),
    )
    def gather_kernel(idx_hbm, table_hbm, out_hbm, idx_v, *bufs_and_sems):
        rows = bufs_and_sems[:_NBUF]
        sem_g = bufs_and_sems[_NBUF:2 * _NBUF]
        sem_o = bufs_and_sems[2 * _NBUF:]

        wid = lax.axis_index("s") * _NC + lax.axis_index("c")
        base = wid * _ROWS_PER_W
        pltpu.sync_copy(idx_hbm.at[wid], idx_v)

        def gather_start(chunk, b):
            pltpu.async_copy(table_hbm.at[idx_v.at[chunk]], rows[b], sem_g[b])

        def gather_wait(chunk, b):
            pltpu.make_async_copy(
                table_hbm.at[idx_v.at[chunk]], rows[b], sem_g[b]
            ).wait()

        def out_slice(chunk):
            return out_hbm.at[pl.ds(base + chunk * _CHUNK, _CHUNK)]

        for b in range(_NBUF):
            gather_start(b, b)

        def round_body(t, carry):
            for b in range(_NBUF):
                chunk = t * _NBUF + b
                gather_wait(chunk, b)
                pltpu.async_copy(rows[b], out_slice(chunk), sem_o[b])
                nxt = chunk + _NBUF

                @pl.when(nxt < _NCHUNK)
                def _():
                    pltpu.make_async_copy(rows[b], out_slice(chunk), sem_o[b]).wait()
                    gather_start(nxt, b)

            return carry

        lax.fori_loop(0, _NROUND, round_body, 0)

        last = _NCHUNK - _NBUF
        for b in range(_NBUF):
            pltpu.make_async_copy(rows[b], out_slice(last + b), sem_o[b]).wait()

    return gather_kernel


_gather = _make_gather()


@jax.jit
def kernel(x, table):
    batch, hist = x.shape
    idx = x.reshape(_NW, _NCHUNK, _CHUNK)
    out = _gather(idx, table)
    return out.reshape(batch, hist, _D)


# speedup vs baseline: 4.6814x; 1.0001x over previous
import functools

import jax
import jax.numpy as jnp
from jax import lax
from jax.experimental import pallas as pl
from jax.experimental.pallas import tpu as pltpu
from jax.experimental.pallas import tpu_sc as plsc

_INFO = plsc.get_sparse_core_info()
_NC = _INFO.num_cores
_NS = _INFO.num_subcores
_NW = _NC * _NS

_D = 64
_B = 4096 * 50
_CHUNK = 128
_ROWS_PER_W = _B // _NW
_NCHUNK = _ROWS_PER_W // _CHUNK
_NBUF = 10
_DEPTH = 5
_NROUND = _NCHUNK // _NBUF


def _make_gather():
    mesh = plsc.VectorSubcoreMesh(core_axis_name="c", subcore_axis_name="s")

    @functools.partial(
        pl.kernel,
        mesh=mesh,
        out_type=jax.ShapeDtypeStruct((_B, _D), jnp.float32),
        scratch_types=(
            [pltpu.VMEM((_NCHUNK, _CHUNK), jnp.int32)]
            + [pltpu.VMEM((_CHUNK, _D), jnp.float32)] * _NBUF
            + [pltpu.SemaphoreType.DMA] * (2 * _NBUF)
        ),
        compiler_params=pltpu.CompilerParams(use_tc_tiling_on_sc=False),
    )
    def gather_kernel(idx_hbm, table_hbm, out_hbm, idx_v, *bufs_and_sems):
        rows = bufs_and_sems[:_NBUF]
        sem_g = bufs_and_sems[_NBUF:2 * _NBUF]
        sem_o = bufs_and_sems[2 * _NBUF:]

        wid = lax.axis_index("s") * _NC + lax.axis_index("c")
        base = wid * _ROWS_PER_W
        pltpu.sync_copy(idx_hbm.at[wid], idx_v)

        def gather_start(chunk, b):
            pltpu.async_copy(table_hbm.at[idx_v.at[chunk]], rows[b], sem_g[b])

        def gather_wait(chunk, b):
            pltpu.make_async_copy(
                table_hbm.at[idx_v.at[chunk]], rows[b], sem_g[b]
            ).wait()

        def out_slice(chunk):
            return out_hbm.at[pl.ds(base + chunk * _CHUNK, _CHUNK)]

        for c in range(_DEPTH):
            gather_start(c, c)

        def round_body(t, carry):
            for j in range(_NBUF):
                chunk = t * _NBUF + j
                gather_wait(chunk, j)
                pltpu.async_copy(rows[j], out_slice(chunk), sem_o[j])
                nxt = chunk + _DEPTH
                bn = (j + _DEPTH) % _NBUF

                @pl.when(jnp.logical_and(nxt < _NCHUNK, nxt >= _NBUF))
                def _():
                    pltpu.make_async_copy(
                        rows[bn], out_slice(nxt - _NBUF), sem_o[bn]
                    ).wait()

                @pl.when(nxt < _NCHUNK)
                def _():
                    gather_start(nxt, bn)

            return carry

        lax.fori_loop(0, _NROUND, round_body, 0)

        last = _NCHUNK - _NBUF
        for j in range(_NBUF):
            pltpu.make_async_copy(rows[j], out_slice(last + j), sem_o[j]).wait()

    return gather_kernel


_gather = _make_gather()


@jax.jit
def kernel(x, table):
    batch, hist = x.shape
    idx = x.reshape(_NW, _NCHUNK, _CHUNK)
    out = _gather(idx, table)
    return out.reshape(batch, hist, _D)
